# baseline (speedup 1.0000x reference)
import functools

import jax
import jax.numpy as jnp
from jax import lax
from jax.experimental import pallas as pl
from jax.experimental.pallas import tpu as pltpu
from jax.experimental.pallas import tpu_sc as plsc

B, C_U, H, W = 4, 256, 128, 128
R, D_GAT = 2048, 128
N = H * W
C_OUT = C_U + D_GAT

NC, NS, L = 2, 16, 16
NW = NC * NS
GPB = NW // B
CPT = D_GAT // GPB
HCH = 16
PCH = HCH * W
NCHUNK = H // HCH
GPW = W // L

_sc_mesh = plsc.VectorSubcoreMesh(core_axis_name="c", subcore_axis_name="s")


@functools.partial(
    pl.kernel,
    mesh=_sc_mesh,
    compiler_params=pltpu.CompilerParams(
        use_tc_tiling_on_sc=False, needs_layout_passes=False
    ),
    out_type=jax.ShapeDtypeStruct((B, C_OUT, H, W), jnp.float32),
    scratch_types=[
        pltpu.VMEM((R, CPT), jnp.float32),
        pltpu.VMEM((HCH, W), jnp.int32),
        pltpu.VMEM((CPT, HCH, W), jnp.float32),
        pltpu.SemaphoreType.DMA,
    ],
)
def _sc_gather(fg_hbm, idx_hbm, fu_hbm, out_hbm, tbl_v, idx_v, ob_v, cp_sem):
    wid = lax.axis_index("s") * NC + lax.axis_index("c")
    b = wid // GPB
    d0 = (wid % GPB) * CPT
    cu0 = (wid % GPB) * (C_U // GPB)
    cp = pltpu.make_async_copy(
        fu_hbm.at[b, pl.ds(cu0, C_U // GPB)],
        out_hbm.at[b, pl.ds(cu0, C_U // GPB)],
        cp_sem,
    )
    cp.start()
    pltpu.sync_copy(fg_hbm.at[:, pl.ds(d0, CPT)], tbl_v)

    def chunk_body(ci, carry):
        pltpu.sync_copy(idx_hbm.at[b, pl.ds(ci * HCH, HCH), :], idx_v)

        def grp(j, carry2):
            r = j // GPW
            c0 = (j % GPW) * L
            iv = idx_v[r, pl.ds(c0, L)]
            for d in range(CPT):
                dvec = jnp.full((L,), d, jnp.int32)
                ob_v[d, r, pl.ds(c0, L)] = plsc.load_gather(tbl_v, [iv, dvec])
            return carry2

        lax.fori_loop(0, HCH * GPW, grp, 0, unroll=False)
        pltpu.sync_copy(
            ob_v, out_hbm.at[b, pl.ds(C_U + d0, CPT), pl.ds(ci * HCH, HCH), :]
        )
        return carry

    lax.fori_loop(0, NCHUNK, chunk_body, 0, unroll=False)
    cp.wait()


def kernel(f_u, f_g, region_to_pixel_map):
    idx32 = region_to_pixel_map.astype(jnp.int32)
    return _sc_gather(f_g, idx32, f_u)
